# baseline (speedup 1.0000x reference)
import functools

import jax
import jax.numpy as jnp
from jax import lax
from jax.experimental import pallas as pl
from jax.experimental.pallas import tpu as pltpu
from jax.experimental.pallas import tpu_sc as plsc

VOCAB = 1000000
DIM = 64
MAXLEN = 200
BATCH = 1024

NTQ = 4
TQ = MAXLEN // NTQ
NBB = 8
BB = BATCH // NBB


def _emb_kernel(xT_hbm, tok_hbm, pos_hbm, out_hbm,
                idxc_v, idx2_v, rows_v, out_v, pos_v, sem):
    c = lax.axis_index("c")
    s = lax.axis_index("s")
    wid = s * 2 + c
    t0 = (wid // NBB) * TQ
    b0 = (wid % NBB) * BB

    pltpu.sync_copy(pos_hbm, pos_v)
    pltpu.sync_copy(xT_hbm.at[:, pl.ds(b0, BB)], idxc_v)

    def halve(j, _):
        for m in range(BB // 16):
            sl = pl.ds(m * 16, 16)
            idx2_v[j, sl] = lax.shift_right_logical(idxc_v[t0 + j, sl], 1)
        return 0

    lax.fori_loop(0, TQ, halve, 0)

    iotas = [lax.broadcasted_iota(jnp.int32, (16,), 0) + m * 16
             for m in range(BB // 16)]

    def step(j, _):
        t = t0 + j
        pltpu.async_copy(tok_hbm.at[idx2_v.at[j]], rows_v, sem).wait()

        for m in range(BB // 16):
            msl = pl.ds(m * 16, 16)
            hoffv = (idxc_v[t, msl] & 1) * DIM
            for dd in range(DIM // 16):
                pv = pos_v[t, pl.ds(dd * 16, 16)]
                for l in range(16):
                    d = dd * 16 + l
                    g = plsc.load_gather(rows_v, [iotas[m], hoffv + d])
                    out_v[d, msl] = g + pv[l]

        pltpu.sync_copy(out_v, out_hbm.at[t, :, pl.ds(b0, BB)])
        return 0

    lax.fori_loop(0, TQ, step, 0)


def kernel(x, token_table, pos_table):
    xT = x.T.astype(jnp.int32)
    tok2 = token_table.reshape(VOCAB // 2, 2 * DIM)
    mesh = plsc.VectorSubcoreMesh(core_axis_name="c", subcore_axis_name="s")
    run = functools.partial(
        pl.kernel,
        mesh=mesh,
        out_type=jax.ShapeDtypeStruct((MAXLEN, DIM, BATCH), jnp.float32),
        scratch_types=[
            pltpu.VMEM((MAXLEN, BB), jnp.int32),
            pltpu.VMEM((TQ, BB), jnp.int32),
            pltpu.VMEM((BB, 2 * DIM), jnp.float32),
            pltpu.VMEM((DIM, BB), jnp.float32),
            pltpu.VMEM((MAXLEN, DIM), jnp.float32),
            pltpu.SemaphoreType.DMA,
        ],
        compiler_params=pltpu.CompilerParams(needs_layout_passes=False),
    )(_emb_kernel)
    oT = run(xT, tok2, pos_table)
    return oT.transpose(2, 0, 1)


# speedup vs baseline: 1.0677x; 1.0677x over previous
import functools

import jax
import jax.numpy as jnp
from jax import lax
from jax.experimental import pallas as pl
from jax.experimental.pallas import tpu as pltpu
from jax.experimental.pallas import tpu_sc as plsc

VOCAB = 1000000
DIM = 64
MAXLEN = 200
BATCH = 1024

NTQ = 4
TQ = MAXLEN // NTQ
NBB = 8
BB = BATCH // NBB


def _emb_kernel(xT_hbm, tok_hbm, posT_hbm, out_hbm,
                idx2_v, hoff_v, rows_v, out_v, posT_v, sem):
    c = lax.axis_index("c")
    s = lax.axis_index("s")
    wid = s * 2 + c
    t0 = (wid // NBB) * TQ
    b0 = (wid % NBB) * BB

    pltpu.sync_copy(posT_hbm, posT_v)
    pltpu.sync_copy(xT_hbm.at[:, pl.ds(b0, BB)], rows_v.at[pl.ds(0, MAXLEN)])

    def halve(j, _):
        for m in range(BB // 16):
            sl = pl.ds(m * 16, 16)
            tok = rows_v[t0 + j, sl].astype(jnp.int32)
            idx2_v[j, sl] = lax.shift_right_logical(tok, 1)
            hoff_v[j, sl] = (tok & 1) * DIM
        return 0

    lax.fori_loop(0, TQ, halve, 0)

    iota16 = lax.broadcasted_iota(jnp.int32, (16,), 0)
    iotas = [iota16 + m * 16 for m in range(BB // 16)]

    def fetch(j, buf):
        return pltpu.async_copy(
            tok_hbm.at[idx2_v.at[j]], rows_v.at[pl.ds(buf * BB, BB)], sem)

    fetch(0, 0)

    def step(j, _):
        t = t0 + j
        buf = lax.rem(j, 2)
        rv = rows_v.at[pl.ds(buf * BB, BB)]
        pltpu.make_async_copy(tok_hbm.at[idx2_v.at[j]], rv, sem).wait()

        @pl.when(j + 1 < TQ)
        def _():
            fetch(j + 1, 1 - buf)

        tsplat = jnp.full((16,), t, jnp.int32)
        hoffs = [hoff_v[j, pl.ds(m * 16, 16)] for m in range(BB // 16)]
        for dd in range(DIM // 16):
            pvs = plsc.load_gather(posT_v, [iotas[dd], tsplat])
            ps = [pvs[l] for l in range(16)]
            for m in range(BB // 16):
                msl = pl.ds(m * 16, 16)
                for l in range(16):
                    d = dd * 16 + l
                    g = plsc.load_gather(rv, [iotas[m], hoffs[m] + d])
                    out_v[d, msl] = g + ps[l]

        pltpu.sync_copy(out_v, out_hbm.at[t, :, pl.ds(b0, BB)])
        return 0

    lax.fori_loop(0, TQ, step, 0)


def kernel(x, token_table, pos_table):
    xT = x.astype(jnp.float32).T
    posT = pos_table.T
    tok2 = token_table.reshape(VOCAB // 2, 2 * DIM)
    mesh = plsc.VectorSubcoreMesh(core_axis_name="c", subcore_axis_name="s")
    run = functools.partial(
        pl.kernel,
        mesh=mesh,
        out_type=jax.ShapeDtypeStruct((MAXLEN, DIM, BATCH), jnp.float32),
        scratch_types=[
            pltpu.VMEM((TQ, BB), jnp.int32),
            pltpu.VMEM((TQ, BB), jnp.int32),
            pltpu.VMEM((2 * BB, 2 * DIM), jnp.float32),
            pltpu.VMEM((DIM, BB), jnp.float32),
            pltpu.VMEM((DIM, MAXLEN), jnp.float32),
            pltpu.SemaphoreType.DMA,
        ],
        compiler_params=pltpu.CompilerParams(needs_layout_passes=False),
    )(_emb_kernel)
    oT = run(xT, tok2, posT)
    return oT.transpose(2, 0, 1)
